# baseline (speedup 1.0000x reference)
import functools
import math

import jax
import jax.numpy as jnp
from jax import lax
from jax.experimental import pallas as pl
from jax.experimental.pallas import tpu as pltpu
from jax.experimental.pallas import tpu_sc as plsc

B, N, DG, DF, K = 8, 1024, 256, 256, 16
BLK = 256
NEG = float("-inf")

_NW = 32
_RPW = (B * N) // _NW
_GPW = _RPW // 16


def _rbf16(v):
    u = plsc.bitcast(v, jnp.uint32)
    lsb = (u >> jnp.uint32(16)) & jnp.uint32(1)
    r = (u + jnp.uint32(0x7FFF) + lsb) & jnp.uint32(0xFFFF0000)
    return plsc.bitcast(r, jnp.float32)


def _thresh_sc_body(xyz_hbm, t_hbm, xbuf, tout, sem):
    wid = lax.axis_index("s") * 2 + lax.axis_index("c")
    b = wid // 4
    iota16 = lax.iota(jnp.int32, 16)
    pltpu.async_copy(xyz_hbm.at[b], xbuf, sem).wait()

    def group_body(gi, _):
        r0 = (wid % 4) * _RPW + gi * 16
        z16 = jnp.zeros((16,), jnp.int32)
        one16 = jnp.ones((16,), jnp.int32)
        two16 = one16 + one16
        bx = [plsc.load_gather(xbuf, [z16, jnp.full((16,), r0 + i, jnp.int32)])
              for i in range(16)]
        by = [plsc.load_gather(xbuf, [one16, jnp.full((16,), r0 + i, jnp.int32)])
              for i in range(16)]
        bz = [plsc.load_gather(xbuf, [two16, jnp.full((16,), r0 + i, jnp.int32)])
              for i in range(16)]
        rbx = [_rbf16(v) for v in bx]
        rby = [_rbf16(v) for v in by]
        rbz = [_rbf16(v) for v in bz]
        bxx = [(bx[i] * bx[i] + by[i] * by[i]) + bz[i] * bz[i]
               for i in range(16)]

        def batch_body(j, carry, rbx=rbx, rby=rby, rbz=rbz, bxx=bxx):
            t, rr = carry
            xm = xbuf[0, pl.ds(j * 16, 16)]
            ym = xbuf[1, pl.ds(j * 16, 16)]
            zm = xbuf[2, pl.ds(j * 16, 16)]
            rxm = _rbf16(xm)
            rym = _rbf16(ym)
            rzm = _rbf16(zm)
            xxm = (xm * xm + ym * ym) + zm * zm
            out = []
            rout = []
            for i in range(16):
                cross = (rxm * rbx[i] + rym * rby[i]) + rzm * rbz[i]
                d = (2.0 * cross - bxx[i]) - xxm
                ds_, _ = plsc.sort_key_val(d, d)
                rev = lax.rev(ds_, (0,))
                mg = jnp.maximum(t[i], rev)
                dropped = jnp.minimum(t[i], rev)
                rout.append(jnp.maximum(rr[i], dropped))
                ts, _ = plsc.sort_key_val(mg, mg)
                out.append(ts)
            return (tuple(out), tuple(rout))

        init = (tuple(jnp.full((16,), NEG, jnp.float32) for _ in range(16)),
                tuple(jnp.full((16,), NEG, jnp.float32) for _ in range(16)))
        t, rr = lax.fori_loop(0, N // 16, batch_body, init)
        tv = jnp.zeros((16,), jnp.float32)
        for i in range(16):
            v16 = jnp.min(t[i])
            v17 = jnp.max(rr[i])
            tv = jnp.where(iota16 == i, 0.5 * (v16 + v17), tv)
        tout[gi] = tv
        return 0

    lax.fori_loop(0, _GPW, group_body, 0)
    pltpu.sync_copy(tout, t_hbm.at[pl.ds(wid * _GPW, _GPW)])


@functools.cache
def _thresh_sc():
    return functools.partial(
        pl.kernel,
        out_type=jax.ShapeDtypeStruct((B * N // 16, 16), jnp.float32),
        mesh=plsc.VectorSubcoreMesh(core_axis_name="c",
                                    subcore_axis_name="s"),
        scratch_types=[
            pltpu.VMEM((3, N), jnp.float32),
            pltpu.VMEM((_GPW, 16), jnp.float32),
            pltpu.SemaphoreType.DMA,
        ],
        compiler_params=pltpu.CompilerParams(needs_layout_passes=False),
    )(_thresh_sc_body)


def _qk_kernel(cat_ref, wq_ref, bq_ref, wk_ref, out_ref):
    q = lax.dot_general(
        cat_ref[0], wq_ref[...], (((1,), (1,)), ((), ())),
        precision=lax.Precision.HIGHEST,
        preferred_element_type=jnp.float32) + bq_ref[...]
    qk = jnp.dot(q, wk_ref[...], precision=lax.Precision.HIGHEST,
                 preferred_element_type=jnp.float32)
    out_ref[0] = qk.astype(jnp.bfloat16)


def _attn_kernel(xyz_blk_ref, xyz_ref, thr_ref, qk_ref, cat_ref,
                 feat_ref, out_ref):
    x_blk = xyz_blk_ref[0]
    x_all = xyz_ref[0]
    inner = lax.dot_general(
        x_blk, x_all, (((1,), (1,)), ((), ())),
        preferred_element_type=jnp.float32)
    xxb = jnp.sum(x_blk * x_blk, axis=1, keepdims=True)
    xxa = jnp.sum(x_all * x_all, axis=1)[None, :]
    dist = 2.0 * inner - xxb - xxa
    mask = dist >= thr_ref[0]

    cat16 = cat_ref[0].astype(jnp.bfloat16)
    s = lax.dot_general(
        qk_ref[0], cat16, (((1,), (1,)), ((), ())),
        preferred_element_type=jnp.float32) * (1.0 / math.sqrt(DG))
    s = jnp.where(mask, s, NEG)
    m = jnp.max(s, axis=1, keepdims=True)
    p = jnp.where(mask, jnp.exp(s - m), 0.0)
    denom = jnp.maximum(jnp.sum(p, axis=1), 1e-30)
    o = lax.dot_general(
        feat_ref[0].astype(jnp.bfloat16), p.astype(jnp.bfloat16),
        (((1,), (1,)), ((), ())),
        preferred_element_type=jnp.float32)
    out_ref[0] = o * (1.0 / denom)[None, :]


@jax.jit
def kernel(fp4_xyz, fp4_features, concatenate_features, Wq, bq, Wk, bk):
    del bk
    nb = N // BLK

    thr = _thresh_sc()(jnp.transpose(fp4_xyz, (0, 2, 1)))

    qk = pl.pallas_call(
        _qk_kernel,
        grid=(B,),
        in_specs=[
            pl.BlockSpec((1, N, DG), lambda b: (b, 0, 0)),
            pl.BlockSpec((DG, DG), lambda b: (0, 0)),
            pl.BlockSpec((1, DG), lambda b: (0, 0)),
            pl.BlockSpec((DG, DG), lambda b: (0, 0)),
        ],
        out_specs=pl.BlockSpec((1, N, DG), lambda b: (b, 0, 0)),
        out_shape=jax.ShapeDtypeStruct((B, N, DG), jnp.bfloat16),
    )(concatenate_features, Wq, bq.reshape(1, DG), Wk)

    out = pl.pallas_call(
        _attn_kernel,
        grid=(B, nb),
        in_specs=[
            pl.BlockSpec((1, BLK, 3), lambda b, n: (b, n, 0)),
            pl.BlockSpec((1, N, 3), lambda b, n: (b, 0, 0)),
            pl.BlockSpec((1, BLK, 1), lambda b, n: (b, n, 0)),
            pl.BlockSpec((1, BLK, DG), lambda b, n: (b, n, 0)),
            pl.BlockSpec((1, N, DG), lambda b, n: (b, 0, 0)),
            pl.BlockSpec((1, DF, N), lambda b, n: (b, 0, 0)),
        ],
        out_specs=pl.BlockSpec((1, DF, BLK), lambda b, n: (b, 0, n)),
        out_shape=jax.ShapeDtypeStruct((B, DF, N), jnp.float32),
    )(fp4_xyz, fp4_xyz, thr.reshape(B, N, 1), qk,
      concatenate_features, fp4_features)
    return out


# speedup vs baseline: 1.7803x; 1.7803x over previous
import functools
import math

import jax
import jax.numpy as jnp
from jax import lax
from jax.experimental import pallas as pl
from jax.experimental.pallas import tpu as pltpu
from jax.experimental.pallas import tpu_sc as plsc

B, N, DG, DF, K = 8, 1024, 256, 256, 16
BLK = 256
NEG = float("-inf")

_NW = 32
_RPW = (B * N) // _NW
_GPW = _RPW // 16


def _oddeven_merge(lo, hi, r):
    step = r * 2
    if step < hi - lo:
        yield from _oddeven_merge(lo, hi, step)
        yield from _oddeven_merge(lo + r, hi, step)
        yield from ((i, i + r) for i in range(lo + r, hi - r, step))
    else:
        yield (lo, lo + r)


def _oem_sort(lo, hi):
    if (hi - lo) >= 1:
        mid = lo + ((hi - lo) // 2)
        yield from _oem_sort(lo, mid)
        yield from _oem_sort(mid + 1, hi)
        yield from _oddeven_merge(lo, hi, 1)


_SORT16 = tuple(_oem_sort(0, 15))
_BITONIC16 = tuple((i, i + d) for d in (8, 4, 2, 1)
                   for i in range(16) if (i & d) == 0)


def _dist_kernel(xyz_blk_ref, xyz_ref, out_ref):
    x_blk = xyz_blk_ref[0]
    x_all = xyz_ref[0]
    inner = lax.dot_general(
        x_blk, x_all, (((1,), (1,)), ((), ())),
        preferred_element_type=jnp.float32)
    xxb = jnp.sum(x_blk * x_blk, axis=1, keepdims=True)
    xxa = jnp.sum(x_all * x_all, axis=1)[None, :]
    out_ref[0] = 2.0 * inner - xxb - xxa


def _thresh_sc_body(d_hbm, t_hbm, buf0, buf1, tout, sem0, sem1):
    wid = lax.axis_index("s") * 2 + lax.axis_index("c")
    rowbase = wid * _RPW
    bufs = (buf0, buf1)
    sems = (sem0, sem1)
    iota16 = lax.iota(jnp.int32, 16)

    def src(gi):
        return d_hbm.at[pl.ds(rowbase + gi * 16, 16)]

    cps = {gi: pltpu.async_copy(src(gi), bufs[gi], sems[gi])
           for gi in range(2)}
    for gi in range(_GPW):
        par = gi % 2
        cps[gi].wait()
        bufr = bufs[par]

        def batch_body(j, t, bufr=bufr):
            out = []
            for i in range(16):
                x = bufr[i, pl.ds(j * 16, 16)]
                xs, _ = plsc.sort_key_val(x, x)
                mg = jnp.maximum(t[i], lax.rev(xs, (0,)))
                ts, _ = plsc.sort_key_val(mg, mg)
                out.append(ts)
            return tuple(out)

        init = tuple(jnp.full((16,), NEG, jnp.float32) for _ in range(16))
        t = lax.fori_loop(0, N // 16, batch_body, init)
        tv = jnp.zeros((16,), jnp.float32)
        for i in range(16):
            tv = jnp.where(iota16 == i, jnp.min(t[i]), tv)
        tout[gi] = tv
        if gi + 2 < _GPW:
            cps[gi + 2] = pltpu.async_copy(src(gi + 2), bufs[par], sems[par])
    pltpu.sync_copy(tout, t_hbm.at[pl.ds(wid * _GPW, _GPW)])


_thresh_sc = functools.partial(
    pl.kernel,
    out_type=jax.ShapeDtypeStruct((B * N // 16, 16), jnp.float32),
    mesh=plsc.VectorSubcoreMesh(core_axis_name="c", subcore_axis_name="s"),
    scratch_types=[
        pltpu.VMEM((16, N), jnp.float32),
        pltpu.VMEM((16, N), jnp.float32),
        pltpu.VMEM((_GPW, 16), jnp.float32),
        pltpu.SemaphoreType.DMA,
        pltpu.SemaphoreType.DMA,
    ],
    compiler_params=pltpu.CompilerParams(needs_layout_passes=False),
)(_thresh_sc_body)


def _attn_kernel(xyz_blk_ref, xyz_ref, thr_ref, cat_blk_ref, cat_ref,
                 feat_ref, wq_ref, bq_ref, wk_ref, out_ref):
    x_blk = xyz_blk_ref[0]
    x_all = xyz_ref[0]
    inner = lax.dot_general(
        x_blk, x_all, (((1,), (1,)), ((), ())),
        preferred_element_type=jnp.float32)
    xxb = jnp.sum(x_blk * x_blk, axis=1, keepdims=True)
    xxa = jnp.sum(x_all * x_all, axis=1)[None, :]
    dist = 2.0 * inner - xxb - xxa
    mask = dist >= thr_ref[0]

    c_blk = cat_blk_ref[0]
    q = lax.dot_general(
        c_blk, wq_ref[...], (((1,), (1,)), ((), ())),
        preferred_element_type=jnp.float32) + bq_ref[...]
    qk = jnp.dot(q, wk_ref[...], preferred_element_type=jnp.float32)
    s = lax.dot_general(
        qk, cat_ref[0], (((1,), (1,)), ((), ())),
        preferred_element_type=jnp.float32) * (1.0 / math.sqrt(DG))
    s = jnp.where(mask, s, NEG)
    m = jnp.max(s, axis=1, keepdims=True)
    p = jnp.where(mask, jnp.exp(s - m), 0.0)
    a = p / jnp.sum(p, axis=1, keepdims=True)
    out_ref[0] = lax.dot_general(
        feat_ref[0], a, (((1,), (1,)), ((), ())),
        preferred_element_type=jnp.float32)


@jax.jit
def kernel(fp4_xyz, fp4_features, concatenate_features, Wq, bq, Wk, bk):
    del bk
    nb = N // BLK
    dist = pl.pallas_call(
        _dist_kernel,
        grid=(B, nb),
        in_specs=[
            pl.BlockSpec((1, BLK, 3), lambda b, n: (b, n, 0)),
            pl.BlockSpec((1, N, 3), lambda b, n: (b, 0, 0)),
        ],
        out_specs=pl.BlockSpec((1, BLK, N), lambda b, n: (b, n, 0)),
        out_shape=jax.ShapeDtypeStruct((B, N, N), jnp.float32),
    )(fp4_xyz, fp4_xyz)

    thr = _thresh_sc(dist.reshape(B * N, N))

    out = pl.pallas_call(
        _attn_kernel,
        grid=(B, nb),
        in_specs=[
            pl.BlockSpec((1, BLK, 3), lambda b, n: (b, n, 0)),
            pl.BlockSpec((1, N, 3), lambda b, n: (b, 0, 0)),
            pl.BlockSpec((1, BLK, 1), lambda b, n: (b, n, 0)),
            pl.BlockSpec((1, BLK, DG), lambda b, n: (b, n, 0)),
            pl.BlockSpec((1, N, DG), lambda b, n: (b, 0, 0)),
            pl.BlockSpec((1, DF, N), lambda b, n: (b, 0, 0)),
            pl.BlockSpec((DG, DG), lambda b, n: (0, 0)),
            pl.BlockSpec((1, DG), lambda b, n: (0, 0)),
            pl.BlockSpec((DG, DG), lambda b, n: (0, 0)),
        ],
        out_specs=pl.BlockSpec((1, DF, BLK), lambda b, n: (b, 0, n)),
        out_shape=jax.ShapeDtypeStruct((B, DF, N), jnp.float32),
    )(fp4_xyz, fp4_xyz, thr.reshape(B, N, 1), concatenate_features,
      concatenate_features, fp4_features, Wq, bq.reshape(1, DG), Wk)
    return out
